# baseline (speedup 1.0000x reference)
import functools
import math

import jax
import jax.numpy as jnp
from jax import lax
from jax.experimental import pallas as pl
from jax.experimental.pallas import tpu as pltpu
from jax.experimental.pallas import tpu_sc as plsc

_V = 1000000
_D = 64
_SCALE = math.sqrt(_D)
_L = 16

_INFO = plsc.get_sparse_core_info()
_NC = _INFO.num_cores
_NS = _INFO.num_subcores
_NW = _NC * _NS

_NBLK = _V // 128
_TAIL_COL = _NBLK * 128


def _make_formatter():
    mesh = plsc.VectorSubcoreMesh(core_axis_name="c", subcore_axis_name="s")

    @functools.partial(
        pl.kernel,
        mesh=mesh,
        out_type=jax.ShapeDtypeStruct((_V // 2, 128), jnp.float32),
        scratch_types=[
            pltpu.VMEM((2, _D, 128), jnp.float32),
            pltpu.VMEM((2, _D, 128), jnp.float32),
            pltpu.VMEM((_D, _D), jnp.float32),
            pltpu.VMEM((32, 128), jnp.float32),
            [pltpu.SemaphoreType.DMA] * 2,
            [pltpu.SemaphoreType.DMA] * 2,
        ],
        compiler_params=pltpu.CompilerParams(needs_layout_passes=False),
    )
    def k(tblt_hbm, tl_hbm, slab, outblk, slab64, outblk64, gs, ss):
        w = lax.axis_index("s") * _NC + lax.axis_index("c")
        iota = lax.iota(jnp.int32, _L)
        rows_k = [iota + ((16 * kk) % _D) for kk in range(8)]

        n_iters = 123

        def in_range(j):
            return (j * _NW + w < _NBLK) & (j < 246)

        def col0(j):
            return pl.multiple_of((j * _NW + w) * 128, 128)

        def issue_read(j, b):
            pltpu.async_copy(
                tblt_hbm.at[:, pl.ds(col0(j), 128)], slab.at[b], gs[b]
            )

        @pl.when(in_range(0))
        def _():
            issue_read(0, 0)

        def outer(jo, _):
            for b in range(2):
                j = jo * 2 + b

                @pl.when(in_range(j))
                def _():
                    pltpu.make_async_copy(
                        tblt_hbm.at[:, pl.ds(col0(j), 128)], slab.at[b], gs[b]
                    ).wait()

                    @pl.when(in_range(j + 1))
                    def _():
                        issue_read(j + 1, 1 - b)

                    @pl.when(j >= 2)
                    def _():
                        pltpu.make_async_copy(
                            outblk.at[b], tl_hbm.at[pl.ds(0, _D)], ss[b]
                        ).wait()

                    bvec = jnp.full((_L,), b, jnp.int32)

                    @plsc.parallel_loop(0, _D, step=1, unroll=2)
                    def _(p):
                        c0 = jnp.full((_L,), 2 * p, jnp.int32)
                        c1 = c0 + 1
                        for kk in range(8):
                            cols = c0 if kk < 4 else c1
                            val = plsc.load_gather(slab, [bvec, rows_k[kk], cols])
                            outblk[b, p, pl.ds(16 * kk, 16)] = val * _SCALE

                    pltpu.async_copy(
                        outblk.at[b],
                        tl_hbm.at[pl.ds(pl.multiple_of((j * _NW + w) * _D, _D), _D)],
                        ss[b],
                    )

            return 0

        lax.fori_loop(0, n_iters, outer, 0)

        for b in range(2):
            pltpu.make_async_copy(
                outblk.at[b], tl_hbm.at[pl.ds(0, _D)], ss[b]
            ).wait()

        @pl.when(w == 4)
        def _():
            pltpu.sync_copy(tblt_hbm.at[:, pl.ds(_TAIL_COL, _D)], slab64)

            @plsc.parallel_loop(0, 32, step=1, unroll=2)
            def _(p):
                c0 = jnp.full((_L,), 2 * p, jnp.int32)
                c1 = c0 + 1
                for kk in range(8):
                    cols = c0 if kk < 4 else c1
                    val = plsc.load_gather(slab64, [rows_k[kk], cols])
                    outblk64[p, pl.ds(16 * kk, 16)] = val * _SCALE

            pltpu.sync_copy(outblk64, tl_hbm.at[pl.ds(_TAIL_COL // 2, 32)])

    return k


def _make_gather(seq):
    mesh = plsc.VectorSubcoreMesh(core_axis_name="c", subcore_axis_name="s")

    @functools.partial(
        pl.kernel,
        mesh=mesh,
        out_type=jax.ShapeDtypeStruct((seq, 8, _NW, 8, 128), jnp.float32),
        scratch_types=[
            pltpu.VMEM((seq, 128), jnp.int32),
            pltpu.VMEM((3, 128), jnp.int32),
            pltpu.VMEM((3, 128, 128), jnp.float32),
            pltpu.VMEM((2, 8, 8, 128), jnp.float32),
            [pltpu.SemaphoreType.DMA] * 3,
            [pltpu.SemaphoreType.DMA] * 2,
        ],
        compiler_params=pltpu.CompilerParams(needs_layout_passes=False),
    )
    def k(tl_hbm, xt_hbm, out_hbm, xv, pbuf, gbuf, obuf, gsems, ssems):
        tc = lax.axis_index("s") * _NC + lax.axis_index("c")
        iota = lax.iota(jnp.int32, _L)
        rows_g = [iota + 16 * g for g in range(8)]

        pltpu.sync_copy(xt_hbm.at[:, pl.ds(pl.multiple_of(tc * 128, 128), 128)], xv)

        def prep_and_fire(s, slot):
            for g in range(8):
                v16 = xv[s, pl.ds(16 * g, 16)]
                pbuf[slot, pl.ds(16 * g, 16)] = lax.shift_right_logical(v16, 1)
            pltpu.async_copy(tl_hbm.at[pbuf.at[slot]], gbuf.at[slot], gsems[slot])

        prep_and_fire(0, 0)
        prep_and_fire(1, 1)

        def step(s, slot, bslot):
            pltpu.make_async_copy(
                tl_hbm.at[pbuf.at[slot]], gbuf.at[slot], gsems[slot]
            ).wait()

            @pl.when(s + 2 < seq)
            def _():
                prep_and_fire(s + 2, (slot + 2) % 3)

            @pl.when(s >= 2)
            def _():
                pltpu.make_async_copy(
                    obuf.at[bslot], out_hbm.at[0, :, 0], ssems[bslot]
                ).wait()

            svec = jnp.full((_L,), slot, jnp.int32)
            for g in range(8):
                v16 = xv[s, pl.ds(16 * g, 16)]
                hvec = (v16 & 1) * _D

                @plsc.parallel_loop(0, _D, step=1, unroll=4)
                def _(d):
                    val = plsc.load_gather(gbuf, [svec, rows_g[g], hvec + d])
                    obuf[bslot, d // 8, d % 8, pl.ds(16 * g, 16)] = val

            pltpu.async_copy(obuf.at[bslot], out_hbm.at[s, :, tc], ssems[bslot])

        def outer(so, _):
            for i in range(6):
                step(so * 6 + i, i % 3, i % 2)
            return 0

        lax.fori_loop(0, seq // 6, outer, 0)
        for i in range(seq % 6):
            step(seq - (seq % 6) + i, i % 3, i % 2)

        for b in range(2):
            pltpu.make_async_copy(
                obuf.at[b], out_hbm.at[0, :, 0], ssems[b]
            ).wait()

    return k


def kernel(x, table):
    b, s = x.shape
    tl = _make_formatter()(table.T)
    out5 = _make_gather(s)(tl, x.T.astype(jnp.int32))
    return out5.transpose(2, 4, 0, 1, 3).reshape(b, s, _D)


# speedup vs baseline: 2.2626x; 2.2626x over previous
import functools
import math

import jax
import jax.numpy as jnp
from jax import lax
from jax.experimental import pallas as pl
from jax.experimental.pallas import tpu as pltpu
from jax.experimental.pallas import tpu_sc as plsc

_V = 1000000
_D = 64
_SCALE = math.sqrt(_D)
_L = 16

_INFO = plsc.get_sparse_core_info()
_NC = _INFO.num_cores
_NS = _INFO.num_subcores
_NW = _NC * _NS

_NBLK = _V // 128
_TAIL_COL = _NBLK * 128


def _make_formatter():
    mesh = plsc.VectorSubcoreMesh(core_axis_name="c", subcore_axis_name="s")

    @functools.partial(
        pl.kernel,
        mesh=mesh,
        out_type=jax.ShapeDtypeStruct((_V // 2, 128), jnp.float32),
        scratch_types=[
            pltpu.VMEM((2, _D, 128), jnp.float32),
            pltpu.VMEM((2, _D, 128), jnp.float32),
            pltpu.VMEM((_D, _D), jnp.float32),
            pltpu.VMEM((32, 128), jnp.float32),
            [pltpu.SemaphoreType.DMA] * 2,
            [pltpu.SemaphoreType.DMA] * 2,
        ],
        compiler_params=pltpu.CompilerParams(needs_layout_passes=False),
    )
    def k(tblt_hbm, tl_hbm, slab, outblk, slab64, outblk64, gs, ss):
        w = lax.axis_index("s") * _NC + lax.axis_index("c")
        iota = lax.iota(jnp.int32, _L)
        rows_k = [iota + ((16 * kk) % _D) for kk in range(8)]

        n_iters = 123

        def in_range(j):
            return (j * _NW + w < _NBLK) & (j < 246)

        def col0(j):
            return pl.multiple_of((j * _NW + w) * 128, 128)

        def issue_read(j, b):
            pltpu.async_copy(
                tblt_hbm.at[:, pl.ds(col0(j), 128)], slab.at[b], gs[b]
            )

        @pl.when(in_range(0))
        def _():
            issue_read(0, 0)

        def outer(jo, _):
            for b in range(2):
                j = jo * 2 + b

                @pl.when(in_range(j))
                def _():
                    pltpu.make_async_copy(
                        tblt_hbm.at[:, pl.ds(col0(j), 128)], slab.at[b], gs[b]
                    ).wait()

                    @pl.when(in_range(j + 1))
                    def _():
                        issue_read(j + 1, 1 - b)

                    @pl.when(j >= 2)
                    def _():
                        pltpu.make_async_copy(
                            outblk.at[b], tl_hbm.at[pl.ds(0, _D)], ss[b]
                        ).wait()

                    bvec = jnp.full((_L,), b, jnp.int32)

                    @plsc.parallel_loop(0, _L, step=1, unroll=2)
                    def _(t):
                        rv = (iota + t) & 15
                        for p0 in range(0, _D, _L):
                            prow = iota + p0
                            cpair = 2 * prow
                            for kk in range(8):
                                rows = rv + ((16 * kk) % _D)
                                cols = cpair + (kk // 4)
                                val = plsc.load_gather(slab, [bvec, rows, cols])
                                plsc.store_scatter(
                                    outblk,
                                    [bvec, prow, rv + 16 * kk],
                                    val * _SCALE,
                                )

                    pltpu.async_copy(
                        outblk.at[b],
                        tl_hbm.at[pl.ds(pl.multiple_of((j * _NW + w) * _D, _D), _D)],
                        ss[b],
                    )

            return 0

        lax.fori_loop(0, n_iters, outer, 0)

        for b in range(2):
            pltpu.make_async_copy(
                outblk.at[b], tl_hbm.at[pl.ds(0, _D)], ss[b]
            ).wait()

        @pl.when(w == 4)
        def _():
            pltpu.sync_copy(tblt_hbm.at[:, pl.ds(_TAIL_COL, _D)], slab64)

            @plsc.parallel_loop(0, _L, step=1, unroll=2)
            def _(t):
                rv = (iota + t) & 15
                for p0 in range(0, 32, _L):
                    prow = iota + p0
                    cpair = 2 * prow
                    for kk in range(8):
                        rows = rv + ((16 * kk) % _D)
                        cols = cpair + (kk // 4)
                        val = plsc.load_gather(slab64, [rows, cols])
                        plsc.store_scatter(
                            outblk64, [prow, rv + 16 * kk], val * _SCALE
                        )

            pltpu.sync_copy(outblk64, tl_hbm.at[pl.ds(_TAIL_COL // 2, 32)])

    return k


def _make_gather(seq):
    mesh = plsc.VectorSubcoreMesh(core_axis_name="c", subcore_axis_name="s")

    @functools.partial(
        pl.kernel,
        mesh=mesh,
        out_type=jax.ShapeDtypeStruct((seq, 8, _NW, 8, 128), jnp.float32),
        scratch_types=[
            pltpu.VMEM((seq, 128), jnp.int32),
            pltpu.VMEM((3, 128), jnp.int32),
            pltpu.VMEM((3, 128, 128), jnp.float32),
            pltpu.VMEM((2, 8, 8, 128), jnp.float32),
            [pltpu.SemaphoreType.DMA] * 3,
            [pltpu.SemaphoreType.DMA] * 2,
        ],
        compiler_params=pltpu.CompilerParams(needs_layout_passes=False),
    )
    def k(tl_hbm, xt_hbm, out_hbm, xv, pbuf, gbuf, obuf, gsems, ssems):
        tc = lax.axis_index("s") * _NC + lax.axis_index("c")
        iota = lax.iota(jnp.int32, _L)
        rows_g = [iota + 16 * g for g in range(8)]

        pltpu.sync_copy(xt_hbm.at[:, pl.ds(pl.multiple_of(tc * 128, 128), 128)], xv)

        def prep_and_fire(s, slot):
            for g in range(8):
                v16 = xv[s, pl.ds(16 * g, 16)]
                pbuf[slot, pl.ds(16 * g, 16)] = lax.shift_right_logical(v16, 1)
            pltpu.async_copy(tl_hbm.at[pbuf.at[slot]], gbuf.at[slot], gsems[slot])

        prep_and_fire(0, 0)
        prep_and_fire(1, 1)

        def step(s, slot, bslot):
            pltpu.make_async_copy(
                tl_hbm.at[pbuf.at[slot]], gbuf.at[slot], gsems[slot]
            ).wait()

            @pl.when(s + 2 < seq)
            def _():
                prep_and_fire(s + 2, (slot + 2) % 3)

            @pl.when(s >= 2)
            def _():
                pltpu.make_async_copy(
                    obuf.at[bslot], out_hbm.at[0, :, 0], ssems[bslot]
                ).wait()

            svec = jnp.full((_L,), slot, jnp.int32)
            bvec = jnp.full((_L,), bslot, jnp.int32)
            hv = []
            for g in range(8):
                v16 = xv[s, pl.ds(16 * g, 16)]
                hv.append((v16 & 1) * _D)

            @plsc.parallel_loop(0, _L, step=1, unroll=2)
            def _(t):
                rv = (iota + t) & 15
                for kk in range(4):
                    dvec = rv + 16 * kk
                    trv = lax.shift_right_logical(dvec, 3)
                    rrv = dvec & 7
                    for g in range(8):
                        val = plsc.load_gather(
                            gbuf, [svec, rows_g[g], hv[g] + dvec]
                        )
                        plsc.store_scatter(
                            obuf, [bvec, trv, rrv, rows_g[g]], val
                        )

            pltpu.async_copy(obuf.at[bslot], out_hbm.at[s, :, tc], ssems[bslot])

        def outer(so, _):
            for i in range(6):
                step(so * 6 + i, i % 3, i % 2)
            return 0

        lax.fori_loop(0, seq // 6, outer, 0)
        for i in range(seq % 6):
            step(seq - (seq % 6) + i, i % 3, i % 2)

        for b in range(2):
            pltpu.make_async_copy(
                obuf.at[b], out_hbm.at[0, :, 0], ssems[b]
            ).wait()

    return k


def kernel(x, table):
    b, s = x.shape
    tl = _make_formatter()(table.T)
    out5 = _make_gather(s)(tl, x.T.astype(jnp.int32))
    return out5.transpose(2, 4, 0, 1, 3).reshape(b, s, _D)


# speedup vs baseline: 2.4493x; 1.0825x over previous
import functools
import math

import jax
import jax.numpy as jnp
from jax import lax
from jax.experimental import pallas as pl
from jax.experimental.pallas import tpu as pltpu
from jax.experimental.pallas import tpu_sc as plsc

_V = 1000000
_D = 64
_SCALE = math.sqrt(_D)
_L = 16

_INFO = plsc.get_sparse_core_info()
_NC = _INFO.num_cores
_NS = _INFO.num_subcores
_NW = _NC * _NS

_NBLK = _V // 128
_TAIL_COL = _NBLK * 128


def _make_formatter():
    mesh = plsc.VectorSubcoreMesh(core_axis_name="c", subcore_axis_name="s")

    @functools.partial(
        pl.kernel,
        mesh=mesh,
        out_type=jax.ShapeDtypeStruct((_V // 2, 128), jnp.float32),
        scratch_types=[
            pltpu.VMEM((2, _D, 128), jnp.float32),
            pltpu.VMEM((2, _D, 128), jnp.float32),
            pltpu.VMEM((_D, _D), jnp.float32),
            pltpu.VMEM((32, 128), jnp.float32),
            [pltpu.SemaphoreType.DMA] * 2,
            [pltpu.SemaphoreType.DMA] * 2,
        ],
        compiler_params=pltpu.CompilerParams(needs_layout_passes=False),
    )
    def k(tblt_hbm, tl_hbm, slab, outblk, slab64, outblk64, gs, ss):
        w = lax.axis_index("s") * _NC + lax.axis_index("c")
        iota = lax.iota(jnp.int32, _L)
        rows_k = [iota + ((16 * kk) % _D) for kk in range(8)]

        n_iters = 123

        def in_range(j):
            return (j * _NW + w < _NBLK) & (j < 246)

        def col0(j):
            return pl.multiple_of((j * _NW + w) * 128, 128)

        def issue_read(j, b):
            pltpu.async_copy(
                tblt_hbm.at[:, pl.ds(col0(j), 128)], slab.at[b], gs[b]
            )

        @pl.when(in_range(0))
        def _():
            issue_read(0, 0)

        def outer(jo, _):
            for b in range(2):
                j = jo * 2 + b

                @pl.when(in_range(j))
                def _():
                    pltpu.make_async_copy(
                        tblt_hbm.at[:, pl.ds(col0(j), 128)], slab.at[b], gs[b]
                    ).wait()

                    @pl.when(in_range(j + 1))
                    def _():
                        issue_read(j + 1, 1 - b)

                    @pl.when(j >= 2)
                    def _():
                        pltpu.make_async_copy(
                            outblk.at[b], tl_hbm.at[pl.ds(0, _D)], ss[b]
                        ).wait()

                    bvec = jnp.full((_L,), b, jnp.int32)

                    @plsc.parallel_loop(0, _L, step=1, unroll=2)
                    def _(t):
                        rv = (iota + t) & 15
                        for p0 in range(0, _D, _L):
                            prow = iota + p0
                            cpair = 2 * prow
                            for kk in range(8):
                                rows = rv + ((16 * kk) % _D)
                                cols = cpair + (kk // 4)
                                val = plsc.load_gather(slab, [bvec, rows, cols])
                                plsc.store_scatter(
                                    outblk,
                                    [bvec, prow, rv + 16 * kk],
                                    val * _SCALE,
                                )

                    pltpu.async_copy(
                        outblk.at[b],
                        tl_hbm.at[pl.ds(pl.multiple_of((j * _NW + w) * _D, _D), _D)],
                        ss[b],
                    )

            return 0

        lax.fori_loop(0, n_iters, outer, 0)

        for b in range(2):
            pltpu.make_async_copy(
                outblk.at[b], tl_hbm.at[pl.ds(0, _D)], ss[b]
            ).wait()

        @pl.when(w == 4)
        def _():
            pltpu.sync_copy(tblt_hbm.at[:, pl.ds(_TAIL_COL, _D)], slab64)

            @plsc.parallel_loop(0, _L, step=1, unroll=2)
            def _(t):
                rv = (iota + t) & 15
                for p0 in range(0, 32, _L):
                    prow = iota + p0
                    cpair = 2 * prow
                    for kk in range(8):
                        rows = rv + ((16 * kk) % _D)
                        cols = cpair + (kk // 4)
                        val = plsc.load_gather(slab64, [rows, cols])
                        plsc.store_scatter(
                            outblk64, [prow, rv + 16 * kk], val * _SCALE
                        )

            pltpu.sync_copy(outblk64, tl_hbm.at[pl.ds(_TAIL_COL // 2, 32)])

    return k


def _make_gather(seq):
    mesh = plsc.VectorSubcoreMesh(core_axis_name="c", subcore_axis_name="s")

    @functools.partial(
        pl.kernel,
        mesh=mesh,
        out_type=jax.ShapeDtypeStruct((seq, 8, _NW, 8, 128), jnp.float32),
        scratch_types=[
            pltpu.VMEM((seq, 128), jnp.int32),
            pltpu.VMEM((3, 128, _D), jnp.float32),
            pltpu.VMEM((2, 8, 8, 128), jnp.float32),
            [pltpu.SemaphoreType.DMA] * 3,
            [pltpu.SemaphoreType.DMA] * 2,
        ],
        compiler_params=pltpu.CompilerParams(
            use_tc_tiling_on_sc=False, needs_layout_passes=False
        ),
    )
    def k(tl_hbm, xt_hbm, out_hbm, xv, gbuf, obuf, gsems, ssems):
        tc = lax.axis_index("s") * _NC + lax.axis_index("c")
        iota = lax.iota(jnp.int32, _L)
        rows_g = [iota + 16 * g for g in range(8)]

        pltpu.sync_copy(xt_hbm.at[:, pl.ds(pl.multiple_of(tc * 128, 128), 128)], xv)

        def fire(s, slot):
            pltpu.async_copy(tl_hbm.at[xv.at[s]], gbuf.at[slot], gsems[slot])

        fire(0, 0)
        fire(1, 1)

        def step(s, slot, bslot):
            pltpu.make_async_copy(
                tl_hbm.at[xv.at[s]], gbuf.at[slot], gsems[slot]
            ).wait()

            @pl.when(s + 2 < seq)
            def _():
                fire(s + 2, (slot + 2) % 3)

            @pl.when(s >= 2)
            def _():
                pltpu.make_async_copy(
                    obuf.at[bslot], out_hbm.at[0, :, 0], ssems[bslot]
                ).wait()

            svec = jnp.full((_L,), slot, jnp.int32)
            bvec = jnp.full((_L,), bslot, jnp.int32)

            @plsc.parallel_loop(0, _L, step=1, unroll=2)
            def _(t):
                rv = (iota + t) & 15
                for kk in range(4):
                    dvec = rv + 16 * kk
                    trv = lax.shift_right_logical(dvec, 3)
                    rrv = dvec & 7
                    for g in range(8):
                        val = plsc.load_gather(gbuf, [svec, rows_g[g], dvec])
                        plsc.store_scatter(
                            obuf, [bvec, trv, rrv, rows_g[g]], val
                        )

            pltpu.async_copy(obuf.at[bslot], out_hbm.at[s, :, tc], ssems[bslot])

        def outer(so, _):
            for i in range(6):
                step(so * 6 + i, i % 3, i % 2)
            return 0

        lax.fori_loop(0, seq // 6, outer, 0)
        for i in range(seq % 6):
            step(seq - (seq % 6) + i, i % 3, i % 2)

        for b in range(2):
            pltpu.make_async_copy(
                obuf.at[b], out_hbm.at[0, :, 0], ssems[b]
            ).wait()

    return k


def kernel(x, table):
    b, s = x.shape
    tl = _make_formatter()(table.T).reshape(_V, _D)
    out5 = _make_gather(s)(tl, x.T.astype(jnp.int32))
    return out5.transpose(2, 4, 0, 1, 3).reshape(b, s, _D)


# speedup vs baseline: 2.6178x; 1.0688x over previous
import functools
import math

import jax
import jax.numpy as jnp
from jax import lax
from jax.experimental import pallas as pl
from jax.experimental.pallas import tpu as pltpu
from jax.experimental.pallas import tpu_sc as plsc

_V = 1000000
_D = 64
_SCALE = math.sqrt(_D)
_L = 16

_INFO = plsc.get_sparse_core_info()
_NC = _INFO.num_cores
_NS = _INFO.num_subcores
_NW = _NC * _NS

_NBLK = _V // 128
_TAIL_COL = _NBLK * 128


def _make_formatter():
    mesh = plsc.VectorSubcoreMesh(core_axis_name="c", subcore_axis_name="s")

    @functools.partial(
        pl.kernel,
        mesh=mesh,
        out_type=jax.ShapeDtypeStruct((_V // 2, 128), jnp.float32),
        scratch_types=[
            pltpu.VMEM((2, _D, 128), jnp.float32),
            pltpu.VMEM((2, _D, 128), jnp.float32),
            pltpu.VMEM((_D, _D), jnp.float32),
            pltpu.VMEM((32, 128), jnp.float32),
            [pltpu.SemaphoreType.DMA] * 2,
            [pltpu.SemaphoreType.DMA] * 2,
        ],
        compiler_params=pltpu.CompilerParams(needs_layout_passes=False),
    )
    def k(tblt_hbm, tl_hbm, slab, outblk, slab64, outblk64, gs, ss):
        w = lax.axis_index("s") * _NC + lax.axis_index("c")
        iota = lax.iota(jnp.int32, _L)
        iota7 = iota & 7
        h1 = lax.shift_right_logical(iota, 3)
        h64 = h1 * _D
        cload = 2 * iota7 + h1

        n_iters = 123

        def in_range(j):
            return (j * _NW + w < _NBLK) & (j < 246)

        def col0(j):
            return pl.multiple_of((j * _NW + w) * 128, 128)

        def issue_read(j, b):
            pltpu.async_copy(
                tblt_hbm.at[:, pl.ds(col0(j), 128)], slab.at[b], gs[b]
            )

        @pl.when(in_range(0))
        def _():
            issue_read(0, 0)

        def outer(jo, _):
            for b in range(2):
                j = jo * 2 + b

                @pl.when(in_range(j))
                def _():
                    pltpu.make_async_copy(
                        tblt_hbm.at[:, pl.ds(col0(j), 128)], slab.at[b], gs[b]
                    ).wait()

                    @pl.when(in_range(j + 1))
                    def _():
                        issue_read(j + 1, 1 - b)

                    @pl.when(j >= 2)
                    def _():
                        pltpu.make_async_copy(
                            outblk.at[b], tl_hbm.at[pl.ds(0, _D)], ss[b]
                        ).wait()

                    bvec = jnp.full((_L,), b, jnp.int32)

                    @plsc.parallel_loop(0, _L, step=1, unroll=2)
                    def _(t):
                        rv = (iota + t) & 15
                        for kk in range(4):
                            dvec = rv + 16 * kk
                            colv = dvec + h64
                            for p0 in range(0, _D, 8):
                                val = plsc.load_gather(
                                    slab, [bvec, dvec, cload + 2 * p0]
                                )
                                plsc.store_scatter(
                                    outblk,
                                    [bvec, iota7 + p0, colv],
                                    val * _SCALE,
                                )

                    pltpu.async_copy(
                        outblk.at[b],
                        tl_hbm.at[pl.ds(pl.multiple_of((j * _NW + w) * _D, _D), _D)],
                        ss[b],
                    )

            return 0

        lax.fori_loop(0, n_iters, outer, 0)

        for b in range(2):
            pltpu.make_async_copy(
                outblk.at[b], tl_hbm.at[pl.ds(0, _D)], ss[b]
            ).wait()

        @pl.when(w == 4)
        def _():
            pltpu.sync_copy(tblt_hbm.at[:, pl.ds(_TAIL_COL, _D)], slab64)

            @plsc.parallel_loop(0, _L, step=1, unroll=2)
            def _(t):
                rv = (iota + t) & 15
                for kk in range(4):
                    dvec = rv + 16 * kk
                    colv = dvec + h64
                    for p0 in range(0, 32, 8):
                        val = plsc.load_gather(slab64, [dvec, cload + 2 * p0])
                        plsc.store_scatter(
                            outblk64, [iota7 + p0, colv], val * _SCALE
                        )

            pltpu.sync_copy(outblk64, tl_hbm.at[pl.ds(_TAIL_COL // 2, 32)])

    return k


def _make_gather(seq):
    mesh = plsc.VectorSubcoreMesh(core_axis_name="c", subcore_axis_name="s")

    @functools.partial(
        pl.kernel,
        mesh=mesh,
        out_type=jax.ShapeDtypeStruct((seq, 8, _NW, 8, 128), jnp.float32),
        scratch_types=[
            pltpu.VMEM((seq, 128), jnp.int32),
            pltpu.VMEM((3, 128, _D), jnp.float32),
            pltpu.VMEM((2, 8, 8, 128), jnp.float32),
            [pltpu.SemaphoreType.DMA] * 3,
            [pltpu.SemaphoreType.DMA] * 2,
        ],
        compiler_params=pltpu.CompilerParams(
            use_tc_tiling_on_sc=False, needs_layout_passes=False
        ),
    )
    def k(tl_hbm, xt_hbm, out_hbm, xv, gbuf, obuf, gsems, ssems):
        tc = lax.axis_index("s") * _NC + lax.axis_index("c")
        iota = lax.iota(jnp.int32, _L)
        rows_g = [iota + 16 * g for g in range(8)]

        pltpu.sync_copy(xt_hbm.at[:, pl.ds(pl.multiple_of(tc * 128, 128), 128)], xv)

        def fire(s, slot):
            pltpu.async_copy(tl_hbm.at[xv.at[s]], gbuf.at[slot], gsems[slot])

        fire(0, 0)
        fire(1, 1)

        def step(s, slot, bslot):
            pltpu.make_async_copy(
                tl_hbm.at[xv.at[s]], gbuf.at[slot], gsems[slot]
            ).wait()

            @pl.when(s + 2 < seq)
            def _():
                fire(s + 2, (slot + 2) % 3)

            @pl.when(s >= 2)
            def _():
                pltpu.make_async_copy(
                    obuf.at[bslot], out_hbm.at[0, :, 0], ssems[bslot]
                ).wait()

            svec = jnp.full((_L,), slot, jnp.int32)
            bvec = jnp.full((_L,), bslot, jnp.int32)

            @plsc.parallel_loop(0, _L, step=1, unroll=2)
            def _(t):
                rv = (iota + t) & 15
                for kk in range(4):
                    dvec = rv + 16 * kk
                    trv = lax.shift_right_logical(dvec, 3)
                    rrv = dvec & 7
                    for g in range(8):
                        val = plsc.load_gather(gbuf, [svec, rows_g[g], dvec])
                        plsc.store_scatter(
                            obuf, [bvec, trv, rrv, rows_g[g]], val
                        )

            pltpu.async_copy(obuf.at[bslot], out_hbm.at[s, :, tc], ssems[bslot])

        def outer(so, _):
            for i in range(6):
                step(so * 6 + i, i % 3, i % 2)
            return 0

        lax.fori_loop(0, seq // 6, outer, 0)
        for i in range(seq % 6):
            step(seq - (seq % 6) + i, i % 3, i % 2)

        for b in range(2):
            pltpu.make_async_copy(
                obuf.at[b], out_hbm.at[0, :, 0], ssems[b]
            ).wait()

    return k


def kernel(x, table):
    b, s = x.shape
    tl = _make_formatter()(table.T).reshape(_V, _D)
    out5 = _make_gather(s)(tl, x.T.astype(jnp.int32))
    return out5.transpose(2, 4, 0, 1, 3).reshape(b, s, _D)


# speedup vs baseline: 3.3804x; 1.2913x over previous
import functools
import math

import jax
import jax.numpy as jnp
from jax import lax
from jax.experimental import pallas as pl
from jax.experimental.pallas import tpu as pltpu
from jax.experimental.pallas import tpu_sc as plsc

_V = 1000000
_D = 64
_SCALE = math.sqrt(_D)
_L = 16

_INFO = plsc.get_sparse_core_info()
_NC = _INFO.num_cores
_NS = _INFO.num_subcores
_NW = _NC * _NS

_NBLK = _V // 128
_TAIL_COL = _NBLK * 128


def _make_formatter():
    mesh = plsc.VectorSubcoreMesh(core_axis_name="c", subcore_axis_name="s")

    @functools.partial(
        pl.kernel,
        mesh=mesh,
        out_type=jax.ShapeDtypeStruct((_V // 2, 128), jnp.float32),
        scratch_types=[
            pltpu.VMEM((2, _D, 256), jnp.float32),
            pltpu.VMEM((2, 128, 128), jnp.float32),
            pltpu.VMEM((_D, _D), jnp.float32),
            pltpu.VMEM((32, 128), jnp.float32),
            [pltpu.SemaphoreType.DMA] * 2,
            [pltpu.SemaphoreType.DMA] * 2,
        ],
        compiler_params=pltpu.CompilerParams(needs_layout_passes=False),
    )
    def k(tblt_hbm, tl_hbm, slab, outblk, slab64, outblk64, gs, ss):
        w = lax.axis_index("s") * _NC + lax.axis_index("c")
        iota = lax.iota(jnp.int32, _L)
        iota7 = iota & 7
        h1 = lax.shift_right_logical(iota, 3)
        h64 = h1 * _D
        cload = 2 * iota7 + h1

        nblk2 = _NBLK // 2

        def in_range(j):
            return j * _NW + w < nblk2

        def col0(j):
            return pl.multiple_of((j * _NW + w) * 256, 256)

        def issue_read(j, slot):
            pltpu.async_copy(
                tblt_hbm.at[:, pl.ds(col0(j), 256)], slab.at[slot], gs[slot]
            )

        @pl.when(in_range(0))
        def _():
            issue_read(0, 0)

        def body(j, slot, bslot):
            @pl.when(in_range(j))
            def _():
                pltpu.make_async_copy(
                    tblt_hbm.at[:, pl.ds(col0(j), 256)], slab.at[slot], gs[slot]
                ).wait()

                @pl.when(in_range(j + 1))
                def _():
                    issue_read(j + 1, 1 - slot)

                @pl.when(j >= 2)
                def _():
                    pltpu.make_async_copy(
                        outblk.at[bslot], tl_hbm.at[pl.ds(0, 128)], ss[bslot]
                    ).wait()

                svec = jnp.full((_L,), slot, jnp.int32)
                bvec = jnp.full((_L,), bslot, jnp.int32)

                @plsc.parallel_loop(0, _L * _L, step=1, unroll=2)
                def _(q):
                    t = q & 15
                    p0 = lax.shift_right_logical(q, 4) * 8
                    rv = (iota + t) & 15
                    prow = iota7 + p0
                    cols = cload + 2 * p0
                    for kk in range(4):
                        dvec = rv + 16 * kk
                        colv = dvec + h64
                        val = plsc.load_gather(slab, [svec, dvec, cols])
                        plsc.store_scatter(
                            outblk, [bvec, prow, colv], val * _SCALE
                        )

                pltpu.async_copy(
                    outblk.at[bslot],
                    tl_hbm.at[
                        pl.ds(pl.multiple_of((j * _NW + w) * 128, 128), 128)
                    ],
                    ss[bslot],
                )

        def outer(jo, _):
            for i in range(2):
                body(jo * 2 + i, i, i)
            return 0

        lax.fori_loop(0, 62, outer, 0)

        for b in range(2):
            pltpu.make_async_copy(
                outblk.at[b], tl_hbm.at[pl.ds(0, 128)], ss[b]
            ).wait()

        @pl.when(w == 4)
        def _():
            pltpu.sync_copy(tblt_hbm.at[:, pl.ds(_TAIL_COL, _D)], slab64)

            @plsc.parallel_loop(0, _L, step=1, unroll=2)
            def _(t):
                rv = (iota + t) & 15
                for kk in range(4):
                    dvec = rv + 16 * kk
                    colv = dvec + h64
                    for p0 in range(0, 32, 8):
                        val = plsc.load_gather(slab64, [dvec, cload + 2 * p0])
                        plsc.store_scatter(
                            outblk64, [iota7 + p0, colv], val * _SCALE
                        )

            pltpu.sync_copy(outblk64, tl_hbm.at[pl.ds(_TAIL_COL // 2, 32)])

    return k


def _make_gather(seq):
    mesh = plsc.VectorSubcoreMesh(core_axis_name="c", subcore_axis_name="s")

    @functools.partial(
        pl.kernel,
        mesh=mesh,
        out_type=jax.ShapeDtypeStruct((seq, 8, _NW, 8, 128), jnp.float32),
        scratch_types=[
            pltpu.VMEM((seq, 128), jnp.int32),
            pltpu.VMEM((3, 128, _D), jnp.float32),
            pltpu.VMEM((2, 8, 8, 128), jnp.float32),
            [pltpu.SemaphoreType.DMA] * 3,
            [pltpu.SemaphoreType.DMA] * 2,
        ],
        compiler_params=pltpu.CompilerParams(
            use_tc_tiling_on_sc=False, needs_layout_passes=False
        ),
    )
    def k(tl_hbm, xt_hbm, out_hbm, xv, gbuf, obuf, gsems, ssems):
        tc = lax.axis_index("s") * _NC + lax.axis_index("c")
        iota = lax.iota(jnp.int32, _L)
        rows_g = [iota + 16 * g for g in range(8)]

        pltpu.sync_copy(xt_hbm.at[:, pl.ds(pl.multiple_of(tc * 128, 128), 128)], xv)

        def fire(s, slot):
            pltpu.async_copy(tl_hbm.at[xv.at[s]], gbuf.at[slot], gsems[slot])

        fire(0, 0)
        fire(1, 1)

        def step(s, slot, bslot):
            pltpu.make_async_copy(
                tl_hbm.at[xv.at[s]], gbuf.at[slot], gsems[slot]
            ).wait()

            @pl.when(s + 2 < seq)
            def _():
                fire(s + 2, (slot + 2) % 3)

            @pl.when(s >= 2)
            def _():
                pltpu.make_async_copy(
                    obuf.at[bslot], out_hbm.at[0, :, 0], ssems[bslot]
                ).wait()

            svec = jnp.full((_L,), slot, jnp.int32)
            bvec = jnp.full((_L,), bslot, jnp.int32)

            @plsc.parallel_loop(0, _L, step=1, unroll=2)
            def _(t):
                rv = (iota + t) & 15
                for kk in range(4):
                    dvec = rv + 16 * kk
                    trv = lax.shift_right_logical(dvec, 3)
                    rrv = dvec & 7
                    for g in range(8):
                        val = plsc.load_gather(gbuf, [svec, rows_g[g], dvec])
                        plsc.store_scatter(
                            obuf, [bvec, trv, rrv, rows_g[g]], val
                        )

            pltpu.async_copy(obuf.at[bslot], out_hbm.at[s, :, tc], ssems[bslot])

        def outer(so, _):
            for i in range(6):
                step(so * 6 + i, i % 3, i % 2)
            return 0

        lax.fori_loop(0, seq // 6, outer, 0)
        for i in range(seq % 6):
            step(seq - (seq % 6) + i, i % 3, i % 2)

        for b in range(2):
            pltpu.make_async_copy(
                obuf.at[b], out_hbm.at[0, :, 0], ssems[b]
            ).wait()

    return k


def kernel(x, table):
    b, s = x.shape
    tl = _make_formatter()(table.T).reshape(_V, _D)
    out5 = _make_gather(s)(tl, x.T.astype(jnp.int32))
    return out5.transpose(2, 4, 0, 1, 3).reshape(b, s, _D)


# speedup vs baseline: 3.8717x; 1.1454x over previous
import functools
import math

import jax
import jax.numpy as jnp
from jax import lax
from jax.experimental import pallas as pl
from jax.experimental.pallas import tpu as pltpu
from jax.experimental.pallas import tpu_sc as plsc

_V = 1000000
_D = 64
_SCALE = math.sqrt(_D)
_L = 16

_INFO = plsc.get_sparse_core_info()
_NC = _INFO.num_cores
_NS = _INFO.num_subcores
_NW = _NC * _NS

_NBLK = _V // 128
_TAIL_COL = _NBLK * 128


def _make_formatter():
    mesh = plsc.VectorSubcoreMesh(core_axis_name="c", subcore_axis_name="s")

    @functools.partial(
        pl.kernel,
        mesh=mesh,
        out_type=jax.ShapeDtypeStruct((_V // 2, 128), jnp.float32),
        scratch_types=[
            pltpu.VMEM((2, _D, 256), jnp.float32),
            pltpu.VMEM((2, 128, 128), jnp.float32),
            pltpu.VMEM((_D, _D), jnp.float32),
            pltpu.VMEM((32, 128), jnp.float32),
            [pltpu.SemaphoreType.DMA] * 2,
            [pltpu.SemaphoreType.DMA] * 2,
        ],
        compiler_params=pltpu.CompilerParams(needs_layout_passes=False),
    )
    def k(tblt_hbm, tl_hbm, slab, outblk, slab64, outblk64, gs, ss):
        w = lax.axis_index("s") * _NC + lax.axis_index("c")
        iota = lax.iota(jnp.int32, _L)
        iota7 = iota & 7
        h1 = lax.shift_right_logical(iota, 3)
        h64 = h1 * _D
        cload = 2 * iota7 + h1

        nblk2 = _NBLK // 2

        def in_range(j):
            return j * _NW + w < nblk2

        def col0(j):
            return pl.multiple_of((j * _NW + w) * 256, 256)

        def issue_read(j, slot):
            pltpu.async_copy(
                tblt_hbm.at[:, pl.ds(col0(j), 256)], slab.at[slot], gs[slot]
            )

        @pl.when(in_range(0))
        def _():
            issue_read(0, 0)

        def body(j, slot, bslot):
            @pl.when(in_range(j))
            def _():
                pltpu.make_async_copy(
                    tblt_hbm.at[:, pl.ds(col0(j), 256)], slab.at[slot], gs[slot]
                ).wait()

                @pl.when(in_range(j + 1))
                def _():
                    issue_read(j + 1, 1 - slot)

                @pl.when(j >= 2)
                def _():
                    pltpu.make_async_copy(
                        outblk.at[bslot], tl_hbm.at[pl.ds(0, 128)], ss[bslot]
                    ).wait()

                svec = jnp.full((_L,), slot, jnp.int32)
                bvec = jnp.full((_L,), bslot, jnp.int32)

                @plsc.parallel_loop(0, _L * _L, step=1, unroll=2)
                def _(q):
                    t = q & 15
                    p0 = lax.shift_right_logical(q, 4) * 8
                    rv = (iota + t) & 15
                    prow = iota7 + p0
                    cols = cload + 2 * p0
                    for kk in range(4):
                        dvec = rv + 16 * kk
                        colv = dvec + h64
                        val = plsc.load_gather(slab, [svec, dvec, cols])
                        plsc.store_scatter(
                            outblk, [bvec, prow, colv], val * _SCALE
                        )

                pltpu.async_copy(
                    outblk.at[bslot],
                    tl_hbm.at[
                        pl.ds(pl.multiple_of((j * _NW + w) * 128, 128), 128)
                    ],
                    ss[bslot],
                )

        def outer(jo, _):
            for i in range(2):
                body(jo * 2 + i, i, i)
            return 0

        lax.fori_loop(0, 62, outer, 0)

        for b in range(2):
            pltpu.make_async_copy(
                outblk.at[b], tl_hbm.at[pl.ds(0, 128)], ss[b]
            ).wait()

        @pl.when(w == 4)
        def _():
            pltpu.sync_copy(tblt_hbm.at[:, pl.ds(_TAIL_COL, _D)], slab64)

            @plsc.parallel_loop(0, _L, step=1, unroll=2)
            def _(t):
                rv = (iota + t) & 15
                for kk in range(4):
                    dvec = rv + 16 * kk
                    colv = dvec + h64
                    for p0 in range(0, 32, 8):
                        val = plsc.load_gather(slab64, [dvec, cload + 2 * p0])
                        plsc.store_scatter(
                            outblk64, [iota7 + p0, colv], val * _SCALE
                        )

            pltpu.sync_copy(outblk64, tl_hbm.at[pl.ds(_TAIL_COL // 2, 32)])

    return k


def _make_gather(seq):
    mesh = plsc.VectorSubcoreMesh(core_axis_name="c", subcore_axis_name="s")

    @functools.partial(
        pl.kernel,
        mesh=mesh,
        out_type=jax.ShapeDtypeStruct((seq, 8, _NW, 8, 128), jnp.float32),
        scratch_types=[
            pltpu.VMEM((seq, 128), jnp.int32),
            pltpu.VMEM((3, 128, _D), jnp.float32),
            pltpu.VMEM((2, 8, 8, 128), jnp.float32),
            [pltpu.SemaphoreType.DMA] * 3,
            [pltpu.SemaphoreType.DMA] * 2,
        ],
        compiler_params=pltpu.CompilerParams(
            use_tc_tiling_on_sc=False, needs_layout_passes=False
        ),
    )
    def k(tl_hbm, xt_hbm, out_hbm, xv, gbuf, obuf, gsems, ssems):
        tc = lax.axis_index("s") * _NC + lax.axis_index("c")
        iota = lax.iota(jnp.int32, _L)
        rows_g = [iota + 16 * g for g in range(8)]

        pltpu.sync_copy(xt_hbm.at[:, pl.ds(pl.multiple_of(tc * 128, 128), 128)], xv)

        def fire(s, slot):
            pltpu.async_copy(tl_hbm.at[xv.at[s]], gbuf.at[slot], gsems[slot])

        fire(0, 0)
        fire(1, 1)

        def step(s, slot, bslot):
            pltpu.make_async_copy(
                tl_hbm.at[xv.at[s]], gbuf.at[slot], gsems[slot]
            ).wait()

            @pl.when(s + 2 < seq)
            def _():
                fire(s + 2, (slot + 2) % 3)

            @pl.when(s >= 2)
            def _():
                pltpu.make_async_copy(
                    obuf.at[bslot], out_hbm.at[0, :, 0], ssems[bslot]
                ).wait()

            svec = jnp.full((_L,), slot, jnp.int32)
            bvec = jnp.full((_L,), bslot, jnp.int32)

            @plsc.parallel_loop(0, _L * 4, step=1, unroll=2)
            def _(q):
                t = q & 15
                kk = lax.shift_right_logical(q, 4)
                dvec = ((iota + t) & 15) + 16 * kk
                trv = lax.shift_right_logical(dvec, 3)
                rrv = dvec & 7
                for g in range(8):
                    val = plsc.load_gather(gbuf, [svec, rows_g[g], dvec])
                    plsc.store_scatter(obuf, [bvec, trv, rrv, rows_g[g]], val)

            pltpu.async_copy(obuf.at[bslot], out_hbm.at[s, :, tc], ssems[bslot])

        def outer(so, _):
            for i in range(6):
                step(so * 6 + i, i % 3, i % 2)
            return 0

        lax.fori_loop(0, seq // 6, outer, 0)
        for i in range(seq % 6):
            step(seq - (seq % 6) + i, i % 3, i % 2)

        for b in range(2):
            pltpu.make_async_copy(
                obuf.at[b], out_hbm.at[0, :, 0], ssems[b]
            ).wait()

    return k


def kernel(x, table):
    b, s = x.shape
    tl = _make_formatter()(table.T).reshape(_V, _D)
    out5 = _make_gather(s)(tl, x.T.astype(jnp.int32))
    return out5.transpose(2, 4, 0, 1, 3).reshape(b, s, _D)
